# baseline (speedup 1.0000x reference)
import jax
import jax.numpy as jnp
from jax import lax
from jax.experimental import pallas as pl
from jax.experimental.pallas import tpu as pltpu
from jax.experimental.pallas import tpu_sc as plsc

NC = 2
NS = 16
NW = NC * NS
BATCH = 16384
D = 32
DB = D // 8
NB = 7813
BPW = (NB + NW - 1) // NW
G = 16
NR = (BPW + G - 1) // G
WL = 1024
SLAB = BATCH // NS


def _scan_table(idx, wl_r, wl_b, blk_lo, blk_hi):
    lane = lax.iota(jnp.int32, 16)
    zero = jnp.zeros((16,), jnp.int32)

    def step(g, cnt):
        v = idx[pl.ds(g * 16, 16)]
        bk = v >> 7
        m = (bk >= blk_lo) & (bk < blk_hi)
        pos = cnt + plsc.cumsum(m.astype(jnp.int32)) - 1
        plsc.store_scatter(wl_r, [pos], v, mask=m)
        plsc.store_scatter(wl_b, [pos], g * 16 + lane, mask=m)
        return cnt + plsc.all_reduce_population_count(m)

    cnt = lax.fori_loop(0, BATCH // 16, step, zero)
    return cnt[0]


def _stream_table(tab, wl_r, wl_b, cnt, rl_r, rl_b, gbuf, rowtmp, plane,
                  blk_lo, gsem):
    lane = lax.iota(jnp.int32, 16)
    sub = lane >> 3
    row8 = lane & 7
    zero = jnp.zeros((16,), jnp.int32)
    nwl = (cnt + 15) >> 4

    def rnd(g, carry):
        gb = jnp.minimum(blk_lo + g * G, NB - G)
        ro = pl.multiple_of(gb * 128, 128)
        cp = pltpu.async_copy(tab.at[:, :, pl.ds(ro, G * 128)], gbuf, gsem)

        def rescan(j, rcnt):
            wv = wl_r[pl.ds(j * 16, 16)]
            bv = wl_b[pl.ds(j * 16, 16)]
            bk = wv >> 7
            m = (bk >= gb) & (bk < gb + G) & (j * 16 + lane < cnt)
            pos = rcnt + plsc.cumsum(m.astype(jnp.int32)) - 1
            plsc.store_scatter(rl_r, [pos], wv, mask=m)
            plsc.store_scatter(rl_b, [pos], bv, mask=m)
            return rcnt + plsc.all_reduce_population_count(m)

        rcnt = lax.fori_loop(0, nwl, rescan, zero)[0]
        cp.wait()

        def extract(k, carry2):
            rv = rl_r[pl.ds(k * 16, 16)]
            bv = rl_b[pl.ds(k * 16, 16)]
            for l in range(16):
                @pl.when(k * 16 + l < rcnt)
                def _():
                    r = rv[l]
                    b = bv[l]
                    col = (((r >> 7) - gb) << 7) + (r & 127)
                    colv = zero + col
                    lo = plsc.load_gather(gbuf, [sub, row8, colv])
                    hi = plsc.load_gather(gbuf, [sub + 2, row8, colv])
                    rowtmp[pl.ds(0, 16)] = lo
                    rowtmp[pl.ds(16, 16)] = hi
                    pltpu.sync_copy(rowtmp, plane.at[pl.ds(b * D, D)])
            return carry2

        lax.fori_loop(0, (rcnt + 15) >> 4, extract, 0)
        return carry

    lax.fori_loop(0, NR, rnd, 0)


def _body(uidx, iidx, tab_u, tab_i, out,
          idx, wl_r, wl_b, rl_r, rl_b, gbuf, rowtmp, zbuf,
          plane, sem, gsem):
    c = lax.axis_index("c")
    s = lax.axis_index("s")
    wid = s * NC + c
    blk_lo = wid * BPW
    blk_hi = jnp.minimum(blk_lo + BPW, NB)

    def zstep(k, carry):
        zbuf[pl.ds(k * 16, 16)] = jnp.zeros((16,), jnp.float32)
        return carry
    lax.fori_loop(0, (SLAB * D // 16) // 16, zstep, 0)

    for t, (idx_hbm, tab_t) in enumerate(((uidx, tab_u), (iidx, tab_i))):
        pltpu.sync_copy(idx_hbm, idx)
        for q in range(16):
            off = (s * SLAB) * D + q * (SLAB * D // 16)
            pltpu.sync_copy(zbuf, plane.at[pl.ds(off, SLAB * D // 16)])
        plsc.subcore_barrier()
        cnt = _scan_table(idx, wl_r, wl_b, blk_lo, blk_hi)
        _stream_table(tab_t, wl_r, wl_b, cnt, rl_r, rl_b, gbuf, rowtmp,
                      plane, blk_lo, gsem)
        plsc.subcore_barrier()
        off = s * SLAB * D
        pltpu.sync_copy(plane.at[pl.ds(off, SLAB * D)],
                        out.at[c, t, pl.ds(off, SLAB * D)])
        plsc.subcore_barrier()


def kernel(x, uid_table, iid_table):
    uidx = x[:, 0]
    iidx = x[:, 1]
    tab_u = uid_table.T.reshape(DB, 8, uid_table.shape[0])
    tab_i = iid_table.T.reshape(DB, 8, iid_table.shape[0])
    f = pl.kernel(
        _body,
        out_type=jax.ShapeDtypeStruct((NC, 2, BATCH * D), jnp.float32),
        mesh=plsc.VectorSubcoreMesh(core_axis_name="c", subcore_axis_name="s"),
        scratch_types=[
            pltpu.VMEM((BATCH,), jnp.int32),
            pltpu.VMEM((WL,), jnp.int32),
            pltpu.VMEM((WL,), jnp.int32),
            pltpu.VMEM((WL,), jnp.int32),
            pltpu.VMEM((WL,), jnp.int32),
            pltpu.VMEM((DB, 8, G * 128), jnp.float32),
            pltpu.VMEM((D,), jnp.float32),
            pltpu.VMEM((SLAB * D // 16,), jnp.float32),
            pltpu.VMEM_SHARED((BATCH * D,), jnp.float32),
            pltpu.SemaphoreType.DMA,
            pltpu.SemaphoreType.DMA,
        ],
        compiler_params=pltpu.CompilerParams(needs_layout_passes=False),
    )
    out = f(uidx, iidx, tab_u, tab_i)
    planes = out[0] + out[1]
    return jnp.transpose(planes.reshape(2, BATCH, D), (1, 0, 2))


# speedup vs baseline: 1.2082x; 1.2082x over previous
import jax
import jax.numpy as jnp
from jax import lax
from jax.experimental import pallas as pl
from jax.experimental.pallas import tpu as pltpu
from jax.experimental.pallas import tpu_sc as plsc

NC = 2
NS = 16
NW = NC * NS
BATCH = 16384
D = 32
DB = D // 8
NB = 7813
BPW = (NB + NW - 1) // NW
G = 8
NR = (BPW + G - 1) // G
WL = 1024
SLAB = BATCH // NS


def _scan_table(idx, wl_r, wl_b, blk_lo, blk_hi):
    lane = lax.iota(jnp.int32, 16)
    zero = jnp.zeros((16,), jnp.int32)

    def step(g, cnt):
        v = idx[pl.ds(g * 16, 16)]
        bk = v >> 7
        m = (bk >= blk_lo) & (bk < blk_hi)
        pos = cnt + plsc.cumsum(m.astype(jnp.int32)) - 1
        plsc.store_scatter(wl_r, [pos], v, mask=m)
        plsc.store_scatter(wl_b, [pos], g * 16 + lane, mask=m)
        return cnt + plsc.all_reduce_population_count(m)

    cnt = lax.fori_loop(0, BATCH // 16, step, zero)
    return cnt[0]


def _gb(blk_lo, g):
    return jnp.minimum(blk_lo + g * G, NB - G)


def _fire(tab, blk_lo, g, buf, sem):
    ro = pl.multiple_of(_gb(blk_lo, g) * 128, 128)
    pltpu.async_copy(tab.at[:, :, pl.ds(ro, G * 128)], buf, sem)


def _stream_table(tab, wl_r, wl_b, cnt, rl_r, rl_b, bufs, sems, rowtmp,
                  plane, blk_lo):
    lane = lax.iota(jnp.int32, 16)
    sub = lane >> 3
    row8 = lane & 7
    zero = jnp.zeros((16,), jnp.int32)
    nwl = (cnt + 15) >> 4

    def consume(g, buf, sem):
        gb = _gb(blk_lo, g)

        def rescan(j, rcnt):
            wv = wl_r[pl.ds(j * 16, 16)]
            bv = wl_b[pl.ds(j * 16, 16)]
            bk = wv >> 7
            m = (bk >= gb) & (bk < gb + G) & (j * 16 + lane < cnt)
            pos = rcnt + plsc.cumsum(m.astype(jnp.int32)) - 1
            plsc.store_scatter(rl_r, [pos], wv, mask=m)
            plsc.store_scatter(rl_b, [pos], bv, mask=m)
            return rcnt + plsc.all_reduce_population_count(m)

        rcnt = lax.fori_loop(0, nwl, rescan, zero)[0]
        pltpu.make_async_copy(tab.at[:, :, pl.ds(0, G * 128)], buf, sem).wait()

        def extract(k, carry2):
            rv = rl_r[pl.ds(k * 16, 16)]
            bv = rl_b[pl.ds(k * 16, 16)]
            for l in range(16):
                @pl.when(k * 16 + l < rcnt)
                def _():
                    r = rv[l]
                    b = bv[l]
                    col = (((r >> 7) - gb) << 7) + (r & 127)
                    colv = zero + col
                    lo = plsc.load_gather(buf, [sub, row8, colv])
                    hi = plsc.load_gather(buf, [sub + 2, row8, colv])
                    rowtmp[pl.ds(0, 16)] = lo
                    rowtmp[pl.ds(16, 16)] = hi
                    pltpu.sync_copy(rowtmp, plane.at[pl.ds(b * D, D)])
            return carry2

        lax.fori_loop(0, (rcnt + 15) >> 4, extract, 0)

    def pair(k, carry):
        _fire(tab, blk_lo, 2 * k + 1, bufs[1], sems[1])
        consume(2 * k, bufs[0], sems[0])
        _fire(tab, blk_lo, 2 * k + 2, bufs[0], sems[0])
        consume(2 * k + 1, bufs[1], sems[1])
        return carry

    lax.fori_loop(0, (NR + 1) // 2, pair, 0)
    pltpu.make_async_copy(
        tab.at[:, :, pl.ds(0, G * 128)], bufs[0], sems[0]).wait()


def _body(uidx, iidx, tab_u, tab_i, out,
          idx, wl_r, wl_b, rl_r, rl_b, gbuf0, gbuf1, rowtmp, zbuf,
          plane, gsem0, gsem1):
    c = lax.axis_index("c")
    s = lax.axis_index("s")
    wid = s * NC + c
    blk_lo = wid * BPW
    blk_hi = jnp.minimum(blk_lo + BPW, NB)

    def zstep(k, carry):
        zbuf[pl.ds(k * 16, 16)] = jnp.zeros((16,), jnp.float32)
        return carry
    lax.fori_loop(0, (SLAB * D // 16) // 16, zstep, 0)

    for t, (idx_hbm, tab_t) in enumerate(((uidx, tab_u), (iidx, tab_i))):
        _fire(tab_t, blk_lo, 0, gbuf0, gsem0)
        pltpu.sync_copy(idx_hbm, idx)
        for q in range(16):
            off = (s * SLAB) * D + q * (SLAB * D // 16)
            pltpu.sync_copy(zbuf, plane.at[pl.ds(off, SLAB * D // 16)])
        plsc.subcore_barrier()
        cnt = _scan_table(idx, wl_r, wl_b, blk_lo, blk_hi)
        _stream_table(tab_t, wl_r, wl_b, cnt, rl_r, rl_b, (gbuf0, gbuf1),
                      (gsem0, gsem1), rowtmp, plane, blk_lo)
        plsc.subcore_barrier()
        off = s * SLAB * D
        pltpu.sync_copy(plane.at[pl.ds(off, SLAB * D)],
                        out.at[c, t, pl.ds(off, SLAB * D)])
        plsc.subcore_barrier()


def kernel(x, uid_table, iid_table):
    uidx = x[:, 0]
    iidx = x[:, 1]
    tab_u = uid_table.T.reshape(DB, 8, uid_table.shape[0])
    tab_i = iid_table.T.reshape(DB, 8, iid_table.shape[0])
    f = pl.kernel(
        _body,
        out_type=jax.ShapeDtypeStruct((NC, 2, BATCH * D), jnp.float32),
        mesh=plsc.VectorSubcoreMesh(core_axis_name="c", subcore_axis_name="s"),
        scratch_types=[
            pltpu.VMEM((BATCH,), jnp.int32),
            pltpu.VMEM((WL,), jnp.int32),
            pltpu.VMEM((WL,), jnp.int32),
            pltpu.VMEM((WL,), jnp.int32),
            pltpu.VMEM((WL,), jnp.int32),
            pltpu.VMEM((DB, 8, G * 128), jnp.float32),
            pltpu.VMEM((DB, 8, G * 128), jnp.float32),
            pltpu.VMEM((D,), jnp.float32),
            pltpu.VMEM((SLAB * D // 16,), jnp.float32),
            pltpu.VMEM_SHARED((BATCH * D,), jnp.float32),
            pltpu.SemaphoreType.DMA,
            pltpu.SemaphoreType.DMA,
        ],
        compiler_params=pltpu.CompilerParams(needs_layout_passes=False),
    )
    out = f(uidx, iidx, tab_u, tab_i)
    planes = out[0] + out[1]
    return jnp.transpose(planes.reshape(2, BATCH, D), (1, 0, 2))


# speedup vs baseline: 1.5626x; 1.2934x over previous
import jax
import jax.numpy as jnp
from jax import lax
from jax.experimental import pallas as pl
from jax.experimental.pallas import tpu as pltpu
from jax.experimental.pallas import tpu_sc as plsc

NC = 2
NS = 16
NW = NC * NS
BATCH = 16384
D = 32
DB = D // 8
NB = 7813
BPW = (NB + NW - 1) // NW
G = 8
NR = (BPW + G - 1) // G
WL = 1024
SLAB = BATCH // NS


def _scan_both(uidx, iidx, ih_u, ih_i, wls, blk_lo, blk_hi):
    lane = lax.iota(jnp.int32, 16)
    zero = jnp.zeros((16,), jnp.int32)
    (wl_ur, wl_ub), (wl_ir, wl_ib) = wls
    cnts = (zero, zero)
    for h in range(2):
        pltpu.sync_copy(uidx.at[pl.ds(h * (BATCH // 2), BATCH // 2)], ih_u)
        pltpu.sync_copy(iidx.at[pl.ds(h * (BATCH // 2), BATCH // 2)], ih_i)

        def step(g, cnt2, h=h):
            cu, ci = cnt2
            vu = ih_u[pl.ds(g * 16, 16)]
            vi = ih_i[pl.ds(g * 16, 16)]
            mu = ((vu >> 7) >= blk_lo) & ((vu >> 7) < blk_hi)
            mi = ((vi >> 7) >= blk_lo) & ((vi >> 7) < blk_hi)
            pu = cu + plsc.cumsum(mu.astype(jnp.int32)) - 1
            pi = ci + plsc.cumsum(mi.astype(jnp.int32)) - 1
            b = h * (BATCH // 2) + g * 16 + lane
            plsc.store_scatter(wl_ur, [pu], vu, mask=mu)
            plsc.store_scatter(wl_ub, [pu], b, mask=mu)
            plsc.store_scatter(wl_ir, [pi], vi, mask=mi)
            plsc.store_scatter(wl_ib, [pi], b, mask=mi)
            return (cu + plsc.all_reduce_population_count(mu),
                    ci + plsc.all_reduce_population_count(mi))

        cnts = lax.fori_loop(0, BATCH // 32, step, cnts)
    return cnts[0][0], cnts[1][0]


def _gb(blk_lo, g):
    return jnp.minimum(blk_lo + g * G, NB - G)


def _fire(tab, blk_lo, g, buf, sem):
    ro = pl.multiple_of(_gb(blk_lo, g) * 128, 128)
    pltpu.async_copy(tab.at[:, :, pl.ds(ro, G * 128)], buf, sem)


def _stream_table(tab, wl_r, wl_b, cnt, rl_r, rl_b, bufs, sems, rowbuf,
                  esem, plane, blk_lo):
    lane = lax.iota(jnp.int32, 16)
    sub = lane >> 3
    row8 = lane & 7
    zero = jnp.zeros((16,), jnp.int32)
    nwl = (cnt + 15) >> 4

    def consume(g, buf, sem):
        gb = _gb(blk_lo, g)

        def rescan(j, rcnt):
            wv = wl_r[pl.ds(j * 16, 16)]
            bv = wl_b[pl.ds(j * 16, 16)]
            bk = wv >> 7
            m = (bk >= gb) & (bk < gb + G) & (j * 16 + lane < cnt)
            pos = rcnt + plsc.cumsum(m.astype(jnp.int32)) - 1
            plsc.store_scatter(rl_r, [pos], wv, mask=m)
            plsc.store_scatter(rl_b, [pos], bv, mask=m)
            return rcnt + plsc.all_reduce_population_count(m)

        rcnt = lax.fori_loop(0, nwl, rescan, zero)[0]
        pltpu.make_async_copy(tab.at[:, :, pl.ds(0, G * 128)], buf, sem).wait()

        def extract(k, carry2):
            rv = rl_r[pl.ds(k * 16, 16)]
            bv = rl_b[pl.ds(k * 16, 16)]
            for l in range(16):
                @pl.when(k * 16 + l < rcnt)
                def _():
                    r = rv[l]
                    b = bv[l]
                    col = (((r >> 7) - gb) << 7) + (r & 127)
                    colv = zero + col
                    lo = plsc.load_gather(buf, [sub, row8, colv])
                    hi = plsc.load_gather(buf, [sub + 2, row8, colv])
                    rowbuf[l, pl.ds(0, 16)] = lo
                    rowbuf[l, pl.ds(16, 16)] = hi
                    pltpu.async_copy(
                        rowbuf.at[l], plane.at[pl.ds(b * D, D)], esem)
            for l in range(16):
                @pl.when(k * 16 + l < rcnt)
                def _():
                    pltpu.make_async_copy(
                        rowbuf.at[l], plane.at[pl.ds(0, D)], esem).wait()
            return carry2

        lax.fori_loop(0, (rcnt + 15) >> 4, extract, 0)

    def pair(k, carry):
        consume(2 * k, bufs[0], sems[0])
        _fire(tab, blk_lo, 2 * k + 2, bufs[0], sems[0])
        consume(2 * k + 1, bufs[1], sems[1])
        _fire(tab, blk_lo, 2 * k + 3, bufs[1], sems[1])
        return carry

    lax.fori_loop(0, (NR + 1) // 2, pair, 0)
    pltpu.make_async_copy(
        tab.at[:, :, pl.ds(0, G * 128)], bufs[0], sems[0]).wait()
    pltpu.make_async_copy(
        tab.at[:, :, pl.ds(0, G * 128)], bufs[1], sems[1]).wait()


def _body(uidx, iidx, tab_u, tab_i, out,
          ih_u, ih_i, wl_ur, wl_ub, wl_ir, wl_ib, rl_r, rl_b,
          gbuf0, gbuf1, rowbuf, zbuf, plane, gsem0, gsem1, esem):
    c = lax.axis_index("c")
    s = lax.axis_index("s")
    wid = s * NC + c
    blk_lo = wid * BPW
    blk_hi = jnp.minimum(blk_lo + BPW, NB)

    def zstep(k, carry):
        zbuf[pl.ds(k * 16, 16)] = jnp.zeros((16,), jnp.float32)
        return carry
    lax.fori_loop(0, (SLAB * D // 16) // 16, zstep, 0)

    def zero_plane():
        for q in range(16):
            off = (s * SLAB) * D + q * (SLAB * D // 16)
            pltpu.sync_copy(zbuf, plane.at[pl.ds(off, SLAB * D // 16)])

    _fire(tab_u, blk_lo, 0, gbuf0, gsem0)
    _fire(tab_u, blk_lo, 1, gbuf1, gsem1)
    zero_plane()
    cnt_u, cnt_i = _scan_both(uidx, iidx, ih_u, ih_i,
                              ((wl_ur, wl_ub), (wl_ir, wl_ib)),
                              blk_lo, blk_hi)
    for t, (tab_t, wr, wb, cnt) in enumerate(
            ((tab_u, wl_ur, wl_ub, cnt_u), (tab_i, wl_ir, wl_ib, cnt_i))):
        if t == 1:
            _fire(tab_t, blk_lo, 0, gbuf0, gsem0)
            _fire(tab_t, blk_lo, 1, gbuf1, gsem1)
            zero_plane()
        plsc.subcore_barrier()
        _stream_table(tab_t, wr, wb, cnt, rl_r, rl_b, (gbuf0, gbuf1),
                      (gsem0, gsem1), rowbuf, esem, plane, blk_lo)
        plsc.subcore_barrier()
        off = s * SLAB * D
        pltpu.sync_copy(plane.at[pl.ds(off, SLAB * D)],
                        out.at[c, t, pl.ds(off, SLAB * D)])
        plsc.subcore_barrier()


def kernel(x, uid_table, iid_table):
    uidx = x[:, 0]
    iidx = x[:, 1]
    tab_u = uid_table.T.reshape(DB, 8, uid_table.shape[0])
    tab_i = iid_table.T.reshape(DB, 8, iid_table.shape[0])
    f = pl.kernel(
        _body,
        out_type=jax.ShapeDtypeStruct((NC, 2, BATCH * D), jnp.float32),
        mesh=plsc.VectorSubcoreMesh(core_axis_name="c", subcore_axis_name="s"),
        scratch_types=[
            pltpu.VMEM((BATCH // 2,), jnp.int32),
            pltpu.VMEM((BATCH // 2,), jnp.int32),
            pltpu.VMEM((WL,), jnp.int32),
            pltpu.VMEM((WL,), jnp.int32),
            pltpu.VMEM((WL,), jnp.int32),
            pltpu.VMEM((WL,), jnp.int32),
            pltpu.VMEM((WL,), jnp.int32),
            pltpu.VMEM((WL,), jnp.int32),
            pltpu.VMEM((DB, 8, G * 128), jnp.float32),
            pltpu.VMEM((DB, 8, G * 128), jnp.float32),
            pltpu.VMEM((16, D), jnp.float32),
            pltpu.VMEM((SLAB * D // 16,), jnp.float32),
            pltpu.VMEM_SHARED((BATCH * D,), jnp.float32),
            pltpu.SemaphoreType.DMA,
            pltpu.SemaphoreType.DMA,
            pltpu.SemaphoreType.DMA,
        ],
        compiler_params=pltpu.CompilerParams(needs_layout_passes=False),
    )
    out = f(uidx, iidx, tab_u, tab_i)
    planes = out[0] + out[1]
    return jnp.transpose(planes.reshape(2, BATCH, D), (1, 0, 2))
